# Initial kernel scaffold; baseline (speedup 1.0000x reference)
import jax
import jax.numpy as jnp
from jax.experimental import pallas as pl


def kernel(x, edge_index, W_in, W_conv1, gamma1, beta1, W_conv2, gamma2, beta2, ngram_weights, W_lin1, b_lin1, W_lin2, b_lin2):
    raise NotImplementedError("write your pallas kernel here")


# speedup vs baseline: 7.4634x; 7.4634x over previous
import functools

import jax
import jax.numpy as jnp
from jax import lax
from jax.experimental import pallas as pl
from jax.experimental.pallas import tpu as pltpu
from jax.experimental.pallas import tpu_sc as plsc

N = 10000
D = 128
NPAD = 10240
NW = 32
CH = 128
ROWS_PER_TILE = NPAD // 16
NGRAM = 6
EPS = 1e-5


def _sc_scatter_body(h_hbm, src_hbm, dst_hbm, zeros_hbm, out_hbm,
                     src_v, dst_v, rows_v, agg_sh, sem):
    c = lax.axis_index("c")
    s = lax.axis_index("s")
    wid = c * 16 + s
    n_chunks = src_v.shape[0]
    base = s * ROWS_PER_TILE

    pltpu.sync_copy(zeros_hbm, agg_sh.at[pl.ds(base, ROWS_PER_TILE)])
    plsc.subcore_barrier()

    pltpu.sync_copy(src_hbm.at[wid], src_v)
    pltpu.sync_copy(dst_hbm.at[wid], dst_v)

    def _chunk(j, carry):
        pltpu.async_copy(h_hbm.at[src_v.at[j]], rows_v, sem).wait()
        pltpu.sync_copy(rows_v, agg_sh.at[dst_v.at[j]], add=True)
        return carry

    lax.fori_loop(0, n_chunks, _chunk, 0)
    plsc.subcore_barrier()

    pltpu.sync_copy(agg_sh.at[pl.ds(base, ROWS_PER_TILE)],
                    out_hbm.at[c, pl.ds(base, ROWS_PER_TILE)])


def _make_sc_scatter(n_chunks):
    mesh = plsc.VectorSubcoreMesh(core_axis_name="c", subcore_axis_name="s")
    return pl.kernel(
        _sc_scatter_body,
        out_type=jax.ShapeDtypeStruct((2, NPAD, D), jnp.float32),
        mesh=mesh,
        scratch_types=[
            pltpu.VMEM((n_chunks, CH), jnp.int32),
            pltpu.VMEM((n_chunks, CH), jnp.int32),
            pltpu.VMEM((CH, D), jnp.float32),
            pltpu.VMEM_SHARED((NPAD, D), jnp.float32),
            pltpu.SemaphoreType.DMA,
        ],
        name="sc_edge_scatter_add",
    )


def _bn_relu(hh, g, b):
    mean = jnp.mean(hh, axis=0, keepdims=True)
    var = jnp.mean(jnp.square(hh - mean), axis=0, keepdims=True)
    y = (hh - mean) * lax.rsqrt(var + EPS) * g + b
    return jnp.maximum(y, 0.0)


def _tc_first_body(agg_ref, Win_ref, W_ref, g_ref, b_ref, h_ref, pool_ref):
    a = agg_ref[0, :N, :] + agg_ref[1, :N, :]
    a = jnp.dot(a, Win_ref[...], preferred_element_type=jnp.float32)
    hh = jnp.dot(a, W_ref[...], preferred_element_type=jnp.float32)
    hr = _bn_relu(hh, g_ref[...], b_ref[...])
    h_ref[...] = hr
    pool_ref[...] = jnp.sum(hr, axis=0, keepdims=True)


def _tc_layer_body(agg_ref, W_ref, g_ref, b_ref, h_ref, pool_ref):
    a = agg_ref[0, :N, :] + agg_ref[1, :N, :]
    hh = jnp.dot(a, W_ref[...], preferred_element_type=jnp.float32)
    hr = _bn_relu(hh, g_ref[...], b_ref[...])
    h_ref[...] = hr
    pool_ref[...] = jnp.sum(hr, axis=0, keepdims=True)


_TC_OUT = [
    jax.ShapeDtypeStruct((N, D), jnp.float32),
    jax.ShapeDtypeStruct((1, D), jnp.float32),
]

_tc_first = pl.pallas_call(_tc_first_body, out_shape=_TC_OUT)
_tc_layer = pl.pallas_call(_tc_layer_body, out_shape=_TC_OUT)


def _head_body(pool_ref, w_ref, W1_ref, b1_ref, W2_ref, b2_ref, out_ref):
    w = jax.nn.softmax(w_ref[...], axis=-1)
    comb = jnp.dot(w, pool_ref[...], preferred_element_type=jnp.float32)
    o = jnp.dot(comb, W1_ref[...], preferred_element_type=jnp.float32)
    o = o + b1_ref[...]
    o = jnp.where(o > 0, o, 0.01 * o)
    o = jnp.dot(o, W2_ref[...], preferred_element_type=jnp.float32)
    o = o + b2_ref[...]
    out_ref[...] = jax.nn.sigmoid(o)


_head = pl.pallas_call(
    _head_body, out_shape=jax.ShapeDtypeStruct((1, 1), jnp.float32))


def _prep_edges(edge_index):
    src = edge_index[0].astype(jnp.int32)
    dst = edge_index[1].astype(jnp.int32)
    e = src.shape[0]
    n_chunks = -(-e // (NW * CH))
    e_pad = NW * n_chunks * CH
    pad = e_pad - e
    ar = jnp.arange(pad, dtype=jnp.int32)
    src_p = jnp.concatenate([src, ar % N]).reshape(NW, n_chunks, CH)
    dst_p = jnp.concatenate([dst, N + ar % (NPAD - N)]).reshape(NW, n_chunks, CH)
    return src_p, dst_p, n_chunks


def _sc_scatter_call(h, src_p, dst_p, zeros_stripe, n_chunks):
    return _make_sc_scatter(n_chunks)(h, src_p, dst_p, zeros_stripe)


def kernel(x, edge_index, W_in, W_conv1, gamma1, beta1, W_conv2, gamma2,
           beta2, ngram_weights, W_lin1, b_lin1, W_lin2, b_lin2):
    src_p, dst_p, n_chunks = _prep_edges(edge_index)
    zeros_stripe = jnp.zeros((ROWS_PER_TILE, D), jnp.float32)

    g1 = gamma1.reshape(1, D)
    b1 = beta1.reshape(1, D)
    g2 = gamma2.reshape(1, D)
    b2 = beta2.reshape(1, D)

    h = x
    pools = []
    for g in range(NGRAM):
        agg = _sc_scatter_call(h, src_p, dst_p, zeros_stripe, n_chunks)
        if g == 0:
            h, _ = _tc_first(agg, W_in, W_conv1, g1, b1)
        else:
            h, _ = _tc_layer(agg, W_conv1, g1, b1)
        agg = _sc_scatter_call(h, src_p, dst_p, zeros_stripe, n_chunks)
        h, pool = _tc_layer(agg, W_conv2, g2, b2)
        pools.append(pool)

    pools8 = jnp.concatenate(pools + [jnp.zeros((2, D), jnp.float32)], axis=0)
    w8 = jnp.concatenate(
        [ngram_weights, jnp.full((2,), -1e30, jnp.float32)]).reshape(1, 8)
    return _head(pools8, w8, W_lin1, b_lin1.reshape(1, -1),
                 W_lin2, b_lin2.reshape(1, -1))


# speedup vs baseline: 9.0660x; 1.2147x over previous
import functools

import jax
import jax.numpy as jnp
from jax import lax
from jax.experimental import pallas as pl
from jax.experimental.pallas import tpu as pltpu
from jax.experimental.pallas import tpu_sc as plsc

N = 10000
D = 128
DH = D // 2
NPAD = 10240
CH = 128
ROWS_PER_TILE = NPAD // 16
NGRAM = 6
EPS = 1e-5


def _sc_scatter_body(h_hbm, sd_hbm, zeros_hbm, out_hbm,
                     sd_v, rows0_v, rows1_v, agg_sh, sem0, sem1):
    c = lax.axis_index("c")
    s = lax.axis_index("s")
    n_chunks = sd_v.shape[1]
    base = s * ROWS_PER_TILE

    pltpu.sync_copy(zeros_hbm, agg_sh.at[pl.ds(base, ROWS_PER_TILE)])
    plsc.subcore_barrier()

    pltpu.sync_copy(sd_hbm.at[c, s], sd_v)

    def _gwait(rows_v, sem):
        pltpu.make_async_copy(h_hbm.at[sd_v.at[0, 0]], rows_v, sem).wait()

    pairs = (n_chunks - 1) // 2
    pltpu.async_copy(h_hbm.at[sd_v.at[0, 0]], rows0_v, sem0)

    def _pair(i, carry):
        j = 2 * i
        pltpu.async_copy(h_hbm.at[sd_v.at[0, j + 1]], rows1_v, sem1)
        _gwait(rows0_v, sem0)
        pltpu.sync_copy(rows0_v, agg_sh.at[sd_v.at[1, j]], add=True)
        pltpu.async_copy(h_hbm.at[sd_v.at[0, j + 2]], rows0_v, sem0)
        _gwait(rows1_v, sem1)
        pltpu.sync_copy(rows1_v, agg_sh.at[sd_v.at[1, j + 1]], add=True)
        return carry

    lax.fori_loop(0, pairs, _pair, 0)
    _gwait(rows0_v, sem0)
    pltpu.sync_copy(rows0_v, agg_sh.at[sd_v.at[1, n_chunks - 1]], add=True)
    plsc.subcore_barrier()

    pltpu.sync_copy(agg_sh.at[pl.ds(base, ROWS_PER_TILE)],
                    out_hbm.at[c, pl.ds(base, ROWS_PER_TILE)])


def _make_sc_scatter(n_chunks):
    mesh = plsc.VectorSubcoreMesh(core_axis_name="c", subcore_axis_name="s")
    return pl.kernel(
        _sc_scatter_body,
        out_type=jax.ShapeDtypeStruct((2, NPAD, DH), jnp.float32),
        mesh=mesh,
        scratch_types=[
            pltpu.VMEM((2, n_chunks, CH), jnp.int32),
            pltpu.VMEM((CH, DH), jnp.float32),
            pltpu.VMEM((CH, DH), jnp.float32),
            pltpu.VMEM_SHARED((NPAD, DH), jnp.float32),
            pltpu.SemaphoreType.DMA,
            pltpu.SemaphoreType.DMA,
        ],
        compiler_params=pltpu.CompilerParams(use_tc_tiling_on_sc=False),
        name="sc_edge_scatter_add",
    )


def _bn_relu(hh, g, b):
    mean = jnp.mean(hh, axis=0, keepdims=True)
    var = jnp.mean(jnp.square(hh - mean), axis=0, keepdims=True)
    y = (hh - mean) * lax.rsqrt(var + EPS) * g + b
    return jnp.maximum(y, 0.0)


def _store_split(h_ref, pool_ref, hr):
    h_ref[0, :, :] = hr[:, :DH]
    h_ref[1, :, :] = hr[:, DH:]
    pool_ref[...] = jnp.sum(hr, axis=0, keepdims=True)


def _tc_first_body(agg_ref, Win_ref, W_ref, g_ref, b_ref, h_ref, pool_ref):
    a = jnp.concatenate([agg_ref[0, :N, :], agg_ref[1, :N, :]], axis=1)
    a = jnp.dot(a, Win_ref[...], preferred_element_type=jnp.float32)
    hh = jnp.dot(a, W_ref[...], preferred_element_type=jnp.float32)
    _store_split(h_ref, pool_ref, _bn_relu(hh, g_ref[...], b_ref[...]))


def _tc_layer_body(agg_ref, W_ref, g_ref, b_ref, h_ref, pool_ref):
    a = jnp.concatenate([agg_ref[0, :N, :], agg_ref[1, :N, :]], axis=1)
    hh = jnp.dot(a, W_ref[...], preferred_element_type=jnp.float32)
    _store_split(h_ref, pool_ref, _bn_relu(hh, g_ref[...], b_ref[...]))


_TC_OUT = [
    jax.ShapeDtypeStruct((2, N, DH), jnp.float32),
    jax.ShapeDtypeStruct((1, D), jnp.float32),
]

_tc_first = pl.pallas_call(_tc_first_body, out_shape=_TC_OUT)
_tc_layer = pl.pallas_call(_tc_layer_body, out_shape=_TC_OUT)


def _head_body(pool_ref, w_ref, W1_ref, b1_ref, W2_ref, b2_ref, out_ref):
    w = jax.nn.softmax(w_ref[...], axis=-1)
    comb = jnp.dot(w, pool_ref[...], preferred_element_type=jnp.float32)
    o = jnp.dot(comb, W1_ref[...], preferred_element_type=jnp.float32)
    o = o + b1_ref[...]
    o = jnp.where(o > 0, o, 0.01 * o)
    o = jnp.dot(o, W2_ref[...], preferred_element_type=jnp.float32)
    o = o + b2_ref[...]
    out_ref[...] = jax.nn.sigmoid(o)


_head = pl.pallas_call(
    _head_body, out_shape=jax.ShapeDtypeStruct((1, 1), jnp.float32))


def _prep_edges(edge_index):
    src = edge_index[0].astype(jnp.int32)
    dst = edge_index[1].astype(jnp.int32)
    e = src.shape[0]
    n_chunks = -(-e // (16 * CH))
    if n_chunks % 2 == 0:
        n_chunks += 1
    e_pad = 16 * n_chunks * CH
    pad = e_pad - e
    ar = jnp.arange(pad, dtype=jnp.int32)
    src_p = jnp.concatenate([src, ar % N]).reshape(1, 16, 1, n_chunks, CH)
    dst_p = jnp.concatenate([dst, N + ar % (NPAD - N)]).reshape(
        1, 16, 1, n_chunks, CH)
    src_2 = jnp.concatenate([src_p, src_p + N], axis=0)
    dst_2 = jnp.concatenate([dst_p, dst_p], axis=0)
    sd_p = jnp.concatenate([src_2, dst_2], axis=2)
    return sd_p, n_chunks


def kernel(x, edge_index, W_in, W_conv1, gamma1, beta1, W_conv2, gamma2,
           beta2, ngram_weights, W_lin1, b_lin1, W_lin2, b_lin2):
    sd_p, n_chunks = _prep_edges(edge_index)
    zeros_stripe = jnp.zeros((ROWS_PER_TILE, DH), jnp.float32)
    sc_scatter = _make_sc_scatter(n_chunks)

    g1 = gamma1.reshape(1, D)
    b1 = beta1.reshape(1, D)
    g2 = gamma2.reshape(1, D)
    b2 = beta2.reshape(1, D)

    h2 = jnp.stack([x[:, :DH], x[:, DH:]])
    pools = []
    for g in range(NGRAM):
        agg = sc_scatter(h2.reshape(2 * N, DH), sd_p, zeros_stripe)
        if g == 0:
            h2, _ = _tc_first(agg, W_in, W_conv1, g1, b1)
        else:
            h2, _ = _tc_layer(agg, W_conv1, g1, b1)
        agg = sc_scatter(h2.reshape(2 * N, DH), sd_p, zeros_stripe)
        h2, pool = _tc_layer(agg, W_conv2, g2, b2)
        pools.append(pool)

    pools8 = jnp.concatenate(pools + [jnp.zeros((2, D), jnp.float32)], axis=0)
    w8 = jnp.concatenate(
        [ngram_weights, jnp.full((2,), -1e30, jnp.float32)]).reshape(1, 8)
    return _head(pools8, w8, W_lin1, b_lin1.reshape(1, -1),
                 W_lin2, b_lin2.reshape(1, -1))


# speedup vs baseline: 10.9158x; 1.2040x over previous
import functools

import jax
import jax.numpy as jnp
from jax import lax
from jax.experimental import pallas as pl
from jax.experimental.pallas import tpu as pltpu
from jax.experimental.pallas import tpu_sc as plsc

N = 10000
D = 128
DH = D // 2
NPAD = 10240
CH = 128
ROWS_PER_TILE = NPAD // 16
NGRAM = 6
EPS = 1e-5


def _sc_scatter_body(h_hbm, sd_hbm, zeros_hbm, out_hbm,
                     sd_v, rows0_v, rows1_v, rows2_v, rows3_v, agg_sh,
                     sem0, sem1, sem2, sem3):
    c = lax.axis_index("c")
    s = lax.axis_index("s")
    n_chunks = sd_v.shape[1]
    base = s * ROWS_PER_TILE

    pltpu.sync_copy(zeros_hbm, agg_sh.at[pl.ds(base, ROWS_PER_TILE)])
    plsc.subcore_barrier()

    pltpu.sync_copy(sd_hbm.at[c, s], sd_v)

    bufs = (rows0_v, rows1_v, rows2_v, rows3_v)
    sems = (sem0, sem1, sem2, sem3)

    def _gstart(j, b):
        pltpu.async_copy(h_hbm.at[sd_v.at[0, j]], bufs[b], sems[b])

    def _gwait(b):
        pltpu.make_async_copy(h_hbm.at[sd_v.at[0, 0]], bufs[b], sems[b]).wait()

    def _scat(j, b):
        pltpu.sync_copy(bufs[b], agg_sh.at[sd_v.at[1, j]], add=True)

    quads = (n_chunks - 3) // 4
    for b in range(3):
        _gstart(b, b)

    def _quad(i, carry):
        j = 4 * i
        for b in range(4):
            _gwait(b)
            _scat(j + b, b)
            _gstart(j + b + 3, (b + 3) % 4)
        return carry

    lax.fori_loop(0, quads, _quad, 0)
    j0 = 4 * quads
    for b in range(3):
        _gwait((j0 + b) % 4)
        _scat(j0 + b, (j0 + b) % 4)
    plsc.subcore_barrier()

    pltpu.sync_copy(agg_sh.at[pl.ds(base, ROWS_PER_TILE)],
                    out_hbm.at[c, pl.ds(base, ROWS_PER_TILE)])


def _make_sc_scatter(n_chunks):
    mesh = plsc.VectorSubcoreMesh(core_axis_name="c", subcore_axis_name="s")
    return pl.kernel(
        _sc_scatter_body,
        out_type=jax.ShapeDtypeStruct((2, NPAD, DH), jnp.float32),
        mesh=mesh,
        scratch_types=[
            pltpu.VMEM((2, n_chunks, CH), jnp.int32),
            pltpu.VMEM((CH, DH), jnp.float32),
            pltpu.VMEM((CH, DH), jnp.float32),
            pltpu.VMEM((CH, DH), jnp.float32),
            pltpu.VMEM((CH, DH), jnp.float32),
            pltpu.VMEM_SHARED((NPAD, DH), jnp.float32),
            pltpu.SemaphoreType.DMA,
            pltpu.SemaphoreType.DMA,
            pltpu.SemaphoreType.DMA,
            pltpu.SemaphoreType.DMA,
        ],
        compiler_params=pltpu.CompilerParams(use_tc_tiling_on_sc=False),
        name="sc_edge_scatter_add",
    )


def _bn_relu(hh, g, b):
    mean = jnp.mean(hh, axis=0, keepdims=True)
    var = jnp.mean(jnp.square(hh - mean), axis=0, keepdims=True)
    y = (hh - mean) * lax.rsqrt(var + EPS) * g + b
    return jnp.maximum(y, 0.0)


def _store_split(h_ref, pool_ref, hr):
    h_ref[0, :, :] = hr[:, :DH]
    h_ref[1, :, :] = hr[:, DH:]
    pool_ref[...] = jnp.sum(hr, axis=0, keepdims=True)


def _tc_first_body(agg_ref, Win_ref, W_ref, g_ref, b_ref, h_ref, pool_ref):
    a = jnp.concatenate([agg_ref[0, :N, :], agg_ref[1, :N, :]], axis=1)
    a = jnp.dot(a, Win_ref[...], preferred_element_type=jnp.float32)
    hh = jnp.dot(a, W_ref[...], preferred_element_type=jnp.float32)
    _store_split(h_ref, pool_ref, _bn_relu(hh, g_ref[...], b_ref[...]))


def _tc_layer_body(agg_ref, W_ref, g_ref, b_ref, h_ref, pool_ref):
    a = jnp.concatenate([agg_ref[0, :N, :], agg_ref[1, :N, :]], axis=1)
    hh = jnp.dot(a, W_ref[...], preferred_element_type=jnp.float32)
    _store_split(h_ref, pool_ref, _bn_relu(hh, g_ref[...], b_ref[...]))


_TC_OUT = [
    jax.ShapeDtypeStruct((2, N, DH), jnp.float32),
    jax.ShapeDtypeStruct((1, D), jnp.float32),
]

_tc_first = pl.pallas_call(_tc_first_body, out_shape=_TC_OUT)
_tc_layer = pl.pallas_call(_tc_layer_body, out_shape=_TC_OUT)


def _head_body(pool_ref, w_ref, W1_ref, b1_ref, W2_ref, b2_ref, out_ref):
    w = jax.nn.softmax(w_ref[...], axis=-1)
    comb = jnp.dot(w, pool_ref[...], preferred_element_type=jnp.float32)
    o = jnp.dot(comb, W1_ref[...], preferred_element_type=jnp.float32)
    o = o + b1_ref[...]
    o = jnp.where(o > 0, o, 0.01 * o)
    o = jnp.dot(o, W2_ref[...], preferred_element_type=jnp.float32)
    o = o + b2_ref[...]
    out_ref[...] = jax.nn.sigmoid(o)


_head = pl.pallas_call(
    _head_body, out_shape=jax.ShapeDtypeStruct((1, 1), jnp.float32))


def _prep_edges(edge_index):
    src = edge_index[0].astype(jnp.int32)
    dst = edge_index[1].astype(jnp.int32)
    e = src.shape[0]
    n_chunks = -(-e // (16 * CH))
    while n_chunks % 4 != 3:
        n_chunks += 1
    e_pad = 16 * n_chunks * CH
    pad = e_pad - e
    ar = jnp.arange(pad, dtype=jnp.int32)
    src_p = jnp.concatenate([src, ar % N]).reshape(1, 16, 1, n_chunks, CH)
    dst_p = jnp.concatenate([dst, N + ar % (NPAD - N)]).reshape(
        1, 16, 1, n_chunks, CH)
    src_2 = jnp.concatenate([src_p, src_p + N], axis=0)
    dst_2 = jnp.concatenate([dst_p, dst_p], axis=0)
    sd_p = jnp.concatenate([src_2, dst_2], axis=2)
    return sd_p, n_chunks


def kernel(x, edge_index, W_in, W_conv1, gamma1, beta1, W_conv2, gamma2,
           beta2, ngram_weights, W_lin1, b_lin1, W_lin2, b_lin2):
    sd_p, n_chunks = _prep_edges(edge_index)
    zeros_stripe = jnp.zeros((ROWS_PER_TILE, DH), jnp.float32)
    sc_scatter = _make_sc_scatter(n_chunks)

    g1 = gamma1.reshape(1, D)
    b1 = beta1.reshape(1, D)
    g2 = gamma2.reshape(1, D)
    b2 = beta2.reshape(1, D)

    h2 = jnp.stack([x[:, :DH], x[:, DH:]])
    pools = []
    for g in range(NGRAM):
        agg = sc_scatter(h2.reshape(2 * N, DH), sd_p, zeros_stripe)
        if g == 0:
            h2, _ = _tc_first(agg, W_in, W_conv1, g1, b1)
        else:
            h2, _ = _tc_layer(agg, W_conv1, g1, b1)
        agg = sc_scatter(h2.reshape(2 * N, DH), sd_p, zeros_stripe)
        h2, pool = _tc_layer(agg, W_conv2, g2, b2)
        pools.append(pool)

    pools8 = jnp.concatenate(pools + [jnp.zeros((2, D), jnp.float32)], axis=0)
    w8 = jnp.concatenate(
        [ngram_weights, jnp.full((2,), -1e30, jnp.float32)]).reshape(1, 8)
    return _head(pools8, w8, W_lin1, b_lin1.reshape(1, -1),
                 W_lin2, b_lin2.reshape(1, -1))


# speedup vs baseline: 48.3931x; 4.4333x over previous
import functools

import jax
import jax.numpy as jnp
from jax import lax
from jax.experimental import pallas as pl
from jax.experimental.pallas import tpu as pltpu
from jax.experimental.pallas import tpu_sc as plsc

N = 10000
D = 128
DH = D // 2
NPAD = 10240
CH = 128
ROWS_PER_TILE = NPAD // 16
NGRAM = 6
EPS = 1e-5


def _sc_scatter_body(h_hbm, sd_hbm, zeros_hbm, out_hbm,
                     sd_v, rows0_v, rows1_v, rows2_v, rows3_v, agg_sh,
                     sem0, sem1, sem2, sem3):
    c = lax.axis_index("c")
    s = lax.axis_index("s")
    n_chunks = sd_v.shape[1]
    base = s * ROWS_PER_TILE

    pltpu.sync_copy(zeros_hbm, agg_sh.at[pl.ds(base, ROWS_PER_TILE)])
    plsc.subcore_barrier()

    pltpu.sync_copy(sd_hbm.at[c, s], sd_v)

    bufs = (rows0_v, rows1_v, rows2_v, rows3_v)
    sems = (sem0, sem1, sem2, sem3)

    def _gstart(j, b):
        pltpu.async_copy(h_hbm.at[sd_v.at[0, j]], bufs[b], sems[b])

    def _gwait(b):
        pltpu.make_async_copy(h_hbm.at[sd_v.at[0, 0]], bufs[b], sems[b]).wait()

    def _scat(j, b):
        pltpu.sync_copy(bufs[b], agg_sh.at[sd_v.at[1, j]], add=True)

    quads = (n_chunks - 3) // 4
    for b in range(3):
        _gstart(b, b)

    def _quad(i, carry):
        j = 4 * i
        for b in range(4):
            _gwait(b)
            _scat(j + b, b)
            _gstart(j + b + 3, (b + 3) % 4)
        return carry

    lax.fori_loop(0, quads, _quad, 0)
    j0 = 4 * quads
    for b in range(3):
        _gwait((j0 + b) % 4)
        _scat(j0 + b, (j0 + b) % 4)
    plsc.subcore_barrier()

    pltpu.sync_copy(agg_sh.at[pl.ds(base, ROWS_PER_TILE)],
                    out_hbm.at[c, pl.ds(base, ROWS_PER_TILE)])


def _make_sc_scatter(n_chunks):
    mesh = plsc.VectorSubcoreMesh(core_axis_name="c", subcore_axis_name="s")
    return pl.kernel(
        _sc_scatter_body,
        out_type=jax.ShapeDtypeStruct((2, NPAD, DH), jnp.float32),
        mesh=mesh,
        scratch_types=[
            pltpu.VMEM((2, n_chunks, CH), jnp.int32),
            pltpu.VMEM((CH, DH), jnp.float32),
            pltpu.VMEM((CH, DH), jnp.float32),
            pltpu.VMEM((CH, DH), jnp.float32),
            pltpu.VMEM((CH, DH), jnp.float32),
            pltpu.VMEM_SHARED((NPAD, DH), jnp.float32),
            pltpu.SemaphoreType.DMA,
            pltpu.SemaphoreType.DMA,
            pltpu.SemaphoreType.DMA,
            pltpu.SemaphoreType.DMA,
        ],
        compiler_params=pltpu.CompilerParams(use_tc_tiling_on_sc=False),
        name="sc_edge_scatter_add",
    )


def _tc_dense(agg_ref, h_ref, pool_ref, g, b, mats):
    p0 = agg_ref[0, : N // 2, :]
    p1 = agg_ref[1, : N // 2, :]
    he = jnp.concatenate([p0[:, :DH], p1[:, :DH]], axis=1)
    ho = jnp.concatenate([p0[:, DH:], p1[:, DH:]], axis=1)
    for m in mats:
        he = jnp.dot(he, m, preferred_element_type=jnp.float32)
        ho = jnp.dot(ho, m, preferred_element_type=jnp.float32)
    mean = (jnp.sum(he, axis=0, keepdims=True) +
            jnp.sum(ho, axis=0, keepdims=True)) / N
    var = (jnp.sum(jnp.square(he - mean), axis=0, keepdims=True) +
           jnp.sum(jnp.square(ho - mean), axis=0, keepdims=True)) / N
    inv = lax.rsqrt(var + EPS)
    he = jnp.maximum((he - mean) * inv * g + b, 0.0)
    ho = jnp.maximum((ho - mean) * inv * g + b, 0.0)
    h_ref[0, :, :] = jnp.concatenate([he[:, :DH], ho[:, :DH]], axis=1)
    h_ref[1, :, :] = jnp.concatenate([he[:, DH:], ho[:, DH:]], axis=1)
    pool_ref[...] = (jnp.sum(he, axis=0, keepdims=True) +
                     jnp.sum(ho, axis=0, keepdims=True))


def _tc_first_body(agg_ref, Win_ref, W_ref, g_ref, b_ref, h_ref, pool_ref):
    _tc_dense(agg_ref, h_ref, pool_ref, g_ref[...], b_ref[...],
              (Win_ref[...], W_ref[...]))


def _tc_layer_body(agg_ref, W_ref, g_ref, b_ref, h_ref, pool_ref):
    _tc_dense(agg_ref, h_ref, pool_ref, g_ref[...], b_ref[...],
              (W_ref[...],))


_TC_OUT = [
    jax.ShapeDtypeStruct((2, N // 2, D), jnp.float32),
    jax.ShapeDtypeStruct((1, D), jnp.float32),
]

_tc_first = pl.pallas_call(_tc_first_body, out_shape=_TC_OUT)
_tc_layer = pl.pallas_call(_tc_layer_body, out_shape=_TC_OUT)


def _head_body(pool_ref, w_ref, W1_ref, b1_ref, W2_ref, b2_ref, out_ref):
    w = jax.nn.softmax(w_ref[...], axis=-1)
    comb = jnp.dot(w, pool_ref[...], preferred_element_type=jnp.float32)
    o = jnp.dot(comb, W1_ref[...], preferred_element_type=jnp.float32)
    o = o + b1_ref[...]
    o = jnp.where(o > 0, o, 0.01 * o)
    o = jnp.dot(o, W2_ref[...], preferred_element_type=jnp.float32)
    o = o + b2_ref[...]
    out_ref[...] = jax.nn.sigmoid(o)


_head = pl.pallas_call(
    _head_body, out_shape=jax.ShapeDtypeStruct((1, 1), jnp.float32))


def _prep_edges(edge_index):
    src = edge_index[0].astype(jnp.int32)
    dst = edge_index[1].astype(jnp.int32)
    e = src.shape[0]
    n_chunks = -(-e // (16 * CH))
    while n_chunks % 4 != 3:
        n_chunks += 1
    e_pad = 16 * n_chunks * CH
    pad = e_pad - e
    ar = jnp.arange(pad, dtype=jnp.int32)
    src_p = (2 * jnp.concatenate([src, ar % N])).reshape(1, 16, 1, n_chunks, CH)
    dst_p = jnp.concatenate([dst, N + ar % (NPAD - N)]).reshape(
        1, 16, 1, n_chunks, CH)
    src_2 = jnp.concatenate([src_p, src_p + 1], axis=0)
    dst_2 = jnp.concatenate([dst_p, dst_p], axis=0)
    sd_p = jnp.concatenate([src_2, dst_2], axis=2)
    return sd_p, n_chunks


def kernel(x, edge_index, W_in, W_conv1, gamma1, beta1, W_conv2, gamma2,
           beta2, ngram_weights, W_lin1, b_lin1, W_lin2, b_lin2):
    sd_p, n_chunks = _prep_edges(edge_index)
    zeros_stripe = jnp.zeros((ROWS_PER_TILE, DH), jnp.float32)
    sc_scatter = _make_sc_scatter(n_chunks)

    g1 = gamma1.reshape(1, D)
    b1 = beta1.reshape(1, D)
    g2 = gamma2.reshape(1, D)
    b2 = beta2.reshape(1, D)

    hflat = x.reshape(2 * N, DH)
    pools = []
    for g in range(NGRAM):
        agg = sc_scatter(hflat, sd_p, zeros_stripe)
        agg128 = agg.reshape(2, NPAD // 2, D)
        if g == 0:
            h2, _ = _tc_first(agg128, W_in, W_conv1, g1, b1)
        else:
            h2, _ = _tc_layer(agg128, W_conv1, g1, b1)
        agg = sc_scatter(h2.reshape(2 * N, DH), sd_p, zeros_stripe)
        agg128 = agg.reshape(2, NPAD // 2, D)
        h2, pool = _tc_layer(agg128, W_conv2, g2, b2)
        pools.append(pool)

    pools8 = jnp.concatenate(pools + [jnp.zeros((2, D), jnp.float32)], axis=0)
    w8 = jnp.concatenate(
        [ngram_weights, jnp.full((2,), -1e30, jnp.float32)]).reshape(1, 8)
    return _head(pools8, w8, W_lin1, b_lin1.reshape(1, -1),
                 W_lin2, b_lin2.reshape(1, -1))
